# baseline (speedup 1.0000x reference)
import functools

import jax
import jax.numpy as jnp
from jax import lax
from jax.experimental import pallas as pl
from jax.experimental.pallas import tpu as pltpu
from jax.experimental.pallas import tpu_sc as plsc

N = 10000
E = 320000
F_IN = 128
HID = 128
C = 64
K = 10

NC = 2
NS = 16
L = 16
CH = C // NC
B = 128
NCHUNK = 158
NSP = 90
EPT = NCHUNK * B
E_PAD = EPT * NS
N_PAD = 10240
NPT = N_PAD // NS


_SPLAT_DN = lax.GatherDimensionNumbers(
    offset_dims=(), collapsed_slice_dims=(0,), start_index_map=(0,))


def _splat_lane(vec, e):
    idx = jnp.full((L, 1), e, jnp.int32)
    return lax.gather(vec, idx, _SPLAT_DN, (1,),
                      mode=lax.GatherScatterMode.PROMISE_IN_BOUNDS)


def _mlp_body(x_ref, w1_ref, b1_ref, g_ref, be_ref, w2_ref, b2_ref, h_ref):
    h1 = jnp.dot(x_ref[...], w1_ref[...], preferred_element_type=jnp.float32)
    h1 = h1 + b1_ref[...][None, :]
    mu = jnp.mean(h1, axis=0, keepdims=True)
    var = jnp.mean((h1 - mu) ** 2, axis=0, keepdims=True)
    hn = (h1 - mu) * lax.rsqrt(var + 1e-5)
    hn = hn * g_ref[...][None, :] + be_ref[...][None, :]
    hr = jnp.maximum(hn, 0.0)
    h = (jnp.dot(hr, w2_ref[...], preferred_element_type=jnp.float32)
         + b2_ref[...][None, :])
    hp = jnp.concatenate(
        [h, jnp.zeros((N_PAD - N, C), jnp.float32)], axis=0)
    h_ref[...] = jnp.stack([hp[:, :CH], hp[:, CH:]], axis=0)


def _mlp(x, W1, b1, gamma, beta, W2, b2):
    return pl.pallas_call(
        _mlp_body,
        out_shape=jax.ShapeDtypeStruct((NC, N_PAD, CH), jnp.float32),
    )(x, W1, b1, gamma, beta, W2, b2)


def _prop_body(hcol, srcs, dsts, norms, out, src_v, dst_v, norm_v, rows_v0,
               rows_v1, zero_v, bufA, bufB, gsem0, gsem1, ssem0, ssem1):
    cid = lax.axis_index("c")
    sid = lax.axis_index("s")

    pltpu.sync_copy(srcs.at[sid], src_v)
    pltpu.sync_copy(dsts.at[sid], dst_v)
    pltpu.sync_copy(norms.at[sid], norm_v)

    pltpu.sync_copy(hcol.at[cid, pl.ds(sid * NPT, NPT)],
                    bufA.at[pl.ds(sid * NPT, NPT)])

    zvec = jnp.zeros((L,), jnp.float32)

    def _zero_row(r, _):
        zero_v[r, pl.ds(0, L)] = zvec
        zero_v[r, pl.ds(L, L)] = zvec
        return 0

    lax.fori_loop(0, NPT, _zero_row, 0)

    rows = (rows_v0, rows_v1)
    gsem = (gsem0, gsem1)
    ssem = (ssem0, ssem1)

    def scale(b, j):
        for g in range(B // L):
            nrm = norm_v[j, pl.ds(g * L, L)]
            for e in range(L):
                sp = _splat_lane(nrm, e)
                r = g * L + e
                rows[b][r, pl.ds(0, L)] = rows[b][r, pl.ds(0, L)] * sp
                rows[b][r, pl.ds(L, L)] = rows[b][r, pl.ds(L, L)] * sp

    def run_pipe(src_of, acc, j0, jend):
        def gather(b, j):
            pltpu.async_copy(src_of.at[src_v.at[j]], rows[b], gsem[b])

        def scatter(b, j):
            pltpu.async_copy(rows[b], acc.at[dst_v.at[j]], ssem[b], add=True)

        def wait_gather(b, j):
            pltpu.make_async_copy(src_of.at[src_v.at[j]], rows[b],
                                  gsem[b]).wait()

        def wait_scatter(b, j):
            pltpu.make_async_copy(rows[b], acc.at[dst_v.at[j]],
                                  ssem[b]).wait()

        gather(0, j0)
        gather(1, j0 + 1)

        def pair(j2, carry):
            j = j0 + 2 * j2
            for b in range(2):
                jj = j + b
                wait_gather(b, jj)
                scale(b, jj)
                scatter(b, jj)
            for b in range(2):
                jn = jnp.minimum(j + 2 + b, jend - 1)
                wait_scatter(b, j + b)
                gather(b, jn)
            return carry

        lax.fori_loop(0, (jend - j0) // 2, pair, 0)
        wait_gather(0, jend - 1)
        wait_gather(1, jend - 1)

    def one_hop(cur, cur_hbm, acc, kidx):
        pltpu.sync_copy(zero_v, acc.at[pl.ds(sid * NPT, NPT)])
        plsc.subcore_barrier()
        run_pipe(cur, acc, 0, NSP)
        run_pipe(cur_hbm, acc, NSP, NCHUNK)
        plsc.subcore_barrier()
        pltpu.sync_copy(acc.at[pl.ds(sid * NPT, NPT)],
                        out.at[cid, kidx, pl.ds(sid * NPT, NPT)])

    one_hop(bufA, hcol.at[cid], bufB, 0)

    def two_hops(i, carry):
        k = 2 * i + 1
        one_hop(bufB, out.at[cid, k - 1], bufA, k)
        one_hop(bufA, out.at[cid, k], bufB, k + 1)
        return carry

    lax.fori_loop(0, (K - 2) // 2, two_hops, 0)
    one_hop(bufB, out.at[cid, K - 2], bufA, K - 1)


def _prop(hcol, srcs, dsts, norms):
    mesh = plsc.VectorSubcoreMesh(core_axis_name="c", subcore_axis_name="s",
                                  num_cores=NC, num_subcores=NS)
    return pl.kernel(
        _prop_body,
        out_type=jax.ShapeDtypeStruct((NC, K, N_PAD, CH), jnp.float32),
        mesh=mesh,
        compiler_params=pltpu.CompilerParams(use_tc_tiling_on_sc=False),
        scratch_types=[
            pltpu.VMEM((NCHUNK, B), jnp.int32),
            pltpu.VMEM((NCHUNK, B), jnp.int32),
            pltpu.VMEM((NCHUNK, B), jnp.float32),
            pltpu.VMEM((B, CH), jnp.float32),
            pltpu.VMEM((B, CH), jnp.float32),
            pltpu.VMEM((NPT, CH), jnp.float32),
            pltpu.VMEM_SHARED((N_PAD, CH), jnp.float32),
            pltpu.VMEM_SHARED((N_PAD, CH), jnp.float32),
            pltpu.SemaphoreType.DMA,
            pltpu.SemaphoreType.DMA,
            pltpu.SemaphoreType.DMA,
            pltpu.SemaphoreType.DMA,
        ],
    )(hcol, srcs, dsts, norms)


ROWS_BLK = 1280


def _final_body(hc_ref, plo_ref, phi_ref, w_ref, bp_ref, out_ref):
    w = w_ref[...][0]
    wlo = w[:CH]
    whi = w[CH:]
    b = bp_ref[0, 0]
    h_lo = hc_ref[0]
    h_hi = hc_ref[1]
    s0 = jax.nn.sigmoid(
        jnp.sum(h_lo * wlo[None, :], axis=1)
        + jnp.sum(h_hi * whi[None, :], axis=1) + b)
    acc_lo = s0[:, None] * h_lo
    acc_hi = s0[:, None] * h_hi
    plo = plo_ref[...]
    phi = phi_ref[...]
    for k in range(K):
        lk = (jnp.sum(plo[k] * wlo[None, :], axis=1)
              + jnp.sum(phi[k] * whi[None, :], axis=1) + b)
        sk = jax.nn.sigmoid(lk)
        acc_lo = acc_lo + sk[:, None] * plo[k]
        acc_hi = acc_hi + sk[:, None] * phi[k]
    out = jnp.concatenate([acc_lo, acc_hi], axis=1)
    m = jnp.max(out, axis=1, keepdims=True)
    ex = jnp.exp(out - m)
    out_ref[...] = out - m - jnp.log(jnp.sum(ex, axis=1, keepdims=True))


def _final(hc, plo, phi, w2d, bp):
    grid = N_PAD // ROWS_BLK
    return pl.pallas_call(
        _final_body,
        grid=(grid,),
        in_specs=[
            pl.BlockSpec((NC, ROWS_BLK, CH), lambda i: (0, i, 0)),
            pl.BlockSpec((K, ROWS_BLK, CH), lambda i: (0, i, 0)),
            pl.BlockSpec((K, ROWS_BLK, CH), lambda i: (0, i, 0)),
            pl.BlockSpec((1, C), lambda i: (0, 0)),
            pl.BlockSpec((1, 1), lambda i: (0, 0)),
        ],
        out_specs=pl.BlockSpec((ROWS_BLK, C), lambda i: (i, 0)),
        out_shape=jax.ShapeDtypeStruct((N_PAD, C), jnp.float32),
        compiler_params=pltpu.CompilerParams(
            vmem_limit_bytes=100 * 1024 * 1024),
    )(hc, plo, phi, w2d, bp)


def kernel(x, edge_index, norm, W1, b1, gamma, beta, W2, b2, w_proj, b_proj):
    hcol = _mlp(x, W1, b1, gamma, beta, W2, b2)

    pad = E_PAD - E
    src = jnp.concatenate([edge_index[0], jnp.zeros((pad,), jnp.int32)])
    dst = jnp.concatenate([edge_index[1], jnp.zeros((pad,), jnp.int32)])
    nrm = jnp.concatenate([norm, jnp.zeros((pad,), jnp.float32)])
    srcs = src.reshape(NS, NCHUNK, B)
    dsts = dst.reshape(NS, NCHUNK, B)
    norms = nrm.reshape(NS, NCHUNK, B)

    preds = _prop(hcol, srcs, dsts, norms)

    out = _final(hcol, preds[0], preds[1], w_proj.reshape(1, C),
                 jnp.reshape(b_proj, (1, 1)))
    return out[:N]


# speedup vs baseline: 1.5771x; 1.5771x over previous
import functools

import jax
import jax.numpy as jnp
from jax import lax
from jax.experimental import pallas as pl
from jax.experimental.pallas import tpu as pltpu
from jax.experimental.pallas import tpu_sc as plsc

N = 10000
E = 320000
F_IN = 128
HID = 128
C = 64
K = 10

NC = 2
NS = 16
L = 16
CH = C // NC
B = 128
NBUF = 2
NCHUNK = 158
EPT = NCHUNK * B
E_PAD = EPT * NS
N_PAD = 10240
NPT = N_PAD // NS


_SPLAT_DN = lax.GatherDimensionNumbers(
    offset_dims=(), collapsed_slice_dims=(0,), start_index_map=(0,))


def _splat_lane(vec, e):
    idx = jnp.full((L, 1), e, jnp.int32)
    return lax.gather(vec, idx, _SPLAT_DN, (1,),
                      mode=lax.GatherScatterMode.PROMISE_IN_BOUNDS)


def _mlp_body(x_ref, w1_ref, b1_ref, g_ref, be_ref, w2_ref, b2_ref, h_ref):
    h1 = jnp.dot(x_ref[...], w1_ref[...], preferred_element_type=jnp.float32)
    h1 = h1 + b1_ref[...][None, :]
    mu = jnp.mean(h1, axis=0, keepdims=True)
    var = jnp.mean((h1 - mu) ** 2, axis=0, keepdims=True)
    hn = (h1 - mu) * lax.rsqrt(var + 1e-5)
    hn = hn * g_ref[...][None, :] + be_ref[...][None, :]
    hr = jnp.maximum(hn, 0.0)
    h = (jnp.dot(hr, w2_ref[...], preferred_element_type=jnp.float32)
         + b2_ref[...][None, :])
    hp = jnp.concatenate(
        [h, jnp.zeros((N_PAD - N, C), jnp.float32)], axis=0)
    h_ref[...] = jnp.stack([hp[:, :CH], hp[:, CH:]], axis=0)


def _mlp(x, W1, b1, gamma, beta, W2, b2):
    return pl.pallas_call(
        _mlp_body,
        out_shape=jax.ShapeDtypeStruct((NC, N_PAD, CH), jnp.float32),
    )(x, W1, b1, gamma, beta, W2, b2)


def _prop_body(hcol, srcs, dsts, norms, out, src_v, dst_v, norm_v, zero_v,
               bufA, bufB, *rows_and_sems):
    rows = rows_and_sems[:NBUF]
    gsem = rows_and_sems[NBUF:2 * NBUF]
    ssem = rows_and_sems[2 * NBUF:3 * NBUF]
    cid = lax.axis_index("c")
    sid = lax.axis_index("s")

    pltpu.sync_copy(srcs.at[sid], src_v)
    pltpu.sync_copy(dsts.at[sid], dst_v)
    pltpu.sync_copy(norms.at[sid], norm_v)

    pltpu.sync_copy(hcol.at[cid, pl.ds(sid * NPT, NPT)],
                    bufA.at[pl.ds(sid * NPT, NPT)])

    zvec = jnp.zeros((L,), jnp.float32)

    def _zero_row(r, _):
        zero_v[r, pl.ds(0, L)] = zvec
        zero_v[r, pl.ds(L, L)] = zvec
        return 0

    lax.fori_loop(0, NPT, _zero_row, 0)

    def scale(b, j):
        for g in range(B // L):
            nrm = norm_v[j, pl.ds(g * L, L)]
            for e in range(L):
                sp = _splat_lane(nrm, e)
                r = g * L + e
                rows[b][r, pl.ds(0, L)] = rows[b][r, pl.ds(0, L)] * sp
                rows[b][r, pl.ds(L, L)] = rows[b][r, pl.ds(L, L)] * sp

    def run_pipe(src_of, acc, j0, jend):
        def gather(b, j):
            pltpu.async_copy(src_of.at[src_v.at[j]], rows[b], gsem[b])

        def scatter(b, j):
            pltpu.async_copy(rows[b], acc.at[dst_v.at[j]], ssem[b], add=True)

        def wait_gather(b, j):
            pltpu.make_async_copy(src_of.at[src_v.at[j]], rows[b],
                                  gsem[b]).wait()

        def wait_scatter(b, j):
            pltpu.make_async_copy(rows[b], acc.at[dst_v.at[j]],
                                  ssem[b]).wait()

        for b in range(NBUF):
            gather(b, j0 + b)

        def quad(jq, carry):
            j = j0 + NBUF * jq
            for b in range(NBUF):
                jj = j + b
                wait_gather(b, jj)
                scale(b, jj)
                scatter(b, jj)
            for b in range(NBUF):
                jn = jnp.minimum(j + NBUF + b, jend - 1)
                wait_scatter(b, j + b)
                gather(b, jn)
            return carry

        lax.fori_loop(0, (jend - j0) // NBUF, quad, 0)
        for b in range(NBUF):
            wait_gather(b, jend - 1)

    def one_hop(cur, acc, kidx):
        pltpu.sync_copy(zero_v, acc.at[pl.ds(sid * NPT, NPT)])
        plsc.subcore_barrier()
        run_pipe(cur, acc, 0, NCHUNK)
        plsc.subcore_barrier()
        pltpu.sync_copy(acc.at[pl.ds(sid * NPT, NPT)],
                        out.at[cid, kidx, pl.ds(sid * NPT, NPT)])

    def two_hops(i, carry):
        one_hop(bufA, bufB, 2 * i)
        one_hop(bufB, bufA, 2 * i + 1)
        return carry

    lax.fori_loop(0, K // 2, two_hops, 0)


def _prop(hcol, srcs, dsts, norms):
    mesh = plsc.VectorSubcoreMesh(core_axis_name="c", subcore_axis_name="s",
                                  num_cores=NC, num_subcores=NS)
    return pl.kernel(
        _prop_body,
        out_type=jax.ShapeDtypeStruct((NC, K, N_PAD, CH), jnp.float32),
        mesh=mesh,
        compiler_params=pltpu.CompilerParams(use_tc_tiling_on_sc=False),
        scratch_types=[
            pltpu.VMEM((NCHUNK, B), jnp.int32),
            pltpu.VMEM((NCHUNK, B), jnp.int32),
            pltpu.VMEM((NCHUNK, B), jnp.float32),
            pltpu.VMEM((NPT, CH), jnp.float32),
            pltpu.VMEM_SHARED((N_PAD, CH), jnp.float32),
            pltpu.VMEM_SHARED((N_PAD, CH), jnp.float32),
        ] + [pltpu.VMEM((B, CH), jnp.float32)] * NBUF
          + [pltpu.SemaphoreType.DMA] * (2 * NBUF),
    )(hcol, srcs, dsts, norms)


ROWS_BLK = 1280


def _final_body(hc_ref, p_ref, w_ref, bp_ref, out_ref):
    w = w_ref[...][0]
    wlo = w[:CH]
    whi = w[CH:]
    b = bp_ref[0, 0]
    h_lo = hc_ref[0]
    h_hi = hc_ref[1]
    plo_ref = p_ref.at[0]
    phi_ref = p_ref.at[1]
    s0 = jax.nn.sigmoid(
        jnp.sum(h_lo * wlo[None, :], axis=1)
        + jnp.sum(h_hi * whi[None, :], axis=1) + b)
    acc_lo = s0[:, None] * h_lo
    acc_hi = s0[:, None] * h_hi
    plo = plo_ref[...]
    phi = phi_ref[...]
    for k in range(K):
        lk = (jnp.sum(plo[k] * wlo[None, :], axis=1)
              + jnp.sum(phi[k] * whi[None, :], axis=1) + b)
        sk = jax.nn.sigmoid(lk)
        acc_lo = acc_lo + sk[:, None] * plo[k]
        acc_hi = acc_hi + sk[:, None] * phi[k]
    out = jnp.concatenate([acc_lo, acc_hi], axis=1)
    m = jnp.max(out, axis=1, keepdims=True)
    ex = jnp.exp(out - m)
    out_ref[...] = out - m - jnp.log(jnp.sum(ex, axis=1, keepdims=True))


def _final(hc, preds, w2d, bp):
    grid = N_PAD // ROWS_BLK
    return pl.pallas_call(
        _final_body,
        grid=(grid,),
        in_specs=[
            pl.BlockSpec((NC, ROWS_BLK, CH), lambda i: (0, i, 0)),
            pl.BlockSpec((NC, K, ROWS_BLK, CH), lambda i: (0, 0, i, 0)),
            pl.BlockSpec((1, C), lambda i: (0, 0)),
            pl.BlockSpec((1, 1), lambda i: (0, 0)),
        ],
        out_specs=pl.BlockSpec((ROWS_BLK, C), lambda i: (i, 0)),
        out_shape=jax.ShapeDtypeStruct((N_PAD, C), jnp.float32),
        compiler_params=pltpu.CompilerParams(
            vmem_limit_bytes=100 * 1024 * 1024),
    )(hc, preds, w2d, bp)


def kernel(x, edge_index, norm, W1, b1, gamma, beta, W2, b2, w_proj, b_proj):
    hcol = _mlp(x, W1, b1, gamma, beta, W2, b2)

    pad = E_PAD - E
    src = jnp.concatenate([edge_index[0], jnp.zeros((pad,), jnp.int32)])
    dst = jnp.concatenate([edge_index[1], jnp.zeros((pad,), jnp.int32)])
    nrm = jnp.concatenate([norm, jnp.zeros((pad,), jnp.float32)])
    srcs = src.reshape(NS, NCHUNK, B)
    dsts = dst.reshape(NS, NCHUNK, B)
    norms = nrm.reshape(NS, NCHUNK, B)

    preds = _prop(hcol, srcs, dsts, norms)

    out = _final(hcol, preds, w_proj.reshape(1, C),
                 jnp.reshape(b_proj, (1, 1)))
    return out[:N]


# speedup vs baseline: 1.6325x; 1.0351x over previous
import functools

import jax
import jax.numpy as jnp
from jax import lax
from jax.experimental import pallas as pl
from jax.experimental.pallas import tpu as pltpu
from jax.experimental.pallas import tpu_sc as plsc

N = 10000
E = 320000
F_IN = 128
HID = 128
C = 64
K = 10

NC = 2
NS = 16
L = 16
CH = C // NC
B = 128
NBUF = 2
NCHUNK = 158
EPT = NCHUNK * B
E_PAD = EPT * NS
N_PAD = 10240
NPT = N_PAD // NS


_SPLAT_DN = lax.GatherDimensionNumbers(
    offset_dims=(), collapsed_slice_dims=(0,), start_index_map=(0,))


def _splat_lane(vec, e):
    idx = jnp.full((L, 1), e, jnp.int32)
    return lax.gather(vec, idx, _SPLAT_DN, (1,),
                      mode=lax.GatherScatterMode.PROMISE_IN_BOUNDS)


def _mlp_body(x_ref, w1_ref, b1_ref, g_ref, be_ref, w2_ref, b2_ref, h_ref):
    h1 = jnp.dot(x_ref[...], w1_ref[...], preferred_element_type=jnp.float32)
    h1 = h1 + b1_ref[...][None, :]
    mu = jnp.mean(h1, axis=0, keepdims=True)
    var = jnp.mean((h1 - mu) ** 2, axis=0, keepdims=True)
    hn = (h1 - mu) * lax.rsqrt(var + 1e-5)
    hn = hn * g_ref[...][None, :] + be_ref[...][None, :]
    hr = jnp.maximum(hn, 0.0)
    h = (jnp.dot(hr, w2_ref[...], preferred_element_type=jnp.float32)
         + b2_ref[...][None, :])
    hp = jnp.concatenate(
        [h, jnp.zeros((N_PAD - N, C), jnp.float32)], axis=0)
    h_ref[...] = jnp.stack([hp[:, :CH], hp[:, CH:]], axis=0)


def _mlp(x, W1, b1, gamma, beta, W2, b2):
    return pl.pallas_call(
        _mlp_body,
        out_shape=jax.ShapeDtypeStruct((NC, N_PAD, CH), jnp.float32),
    )(x, W1, b1, gamma, beta, W2, b2)


def _prop_body(hcol, srcs, dsts, norms, out, src_v, dst_v, norm_v, zero_v,
               bufA, bufB, *rows_and_sems):
    rows = rows_and_sems[:NBUF]
    gsem = rows_and_sems[NBUF:2 * NBUF]
    ssem = rows_and_sems[2 * NBUF:3 * NBUF]
    cid = lax.axis_index("c")
    sid = lax.axis_index("s")

    pltpu.sync_copy(srcs.at[sid], src_v)
    pltpu.sync_copy(dsts.at[sid], dst_v)
    pltpu.sync_copy(norms.at[sid], norm_v)

    pltpu.sync_copy(hcol.at[cid, pl.ds(sid * NPT, NPT)],
                    bufA.at[pl.ds(sid * NPT, NPT)])

    zvec = jnp.zeros((L,), jnp.float32)

    def _zero_row(r, _):
        zero_v[r, pl.ds(0, L)] = zvec
        zero_v[r, pl.ds(L, L)] = zvec
        return 0

    lax.fori_loop(0, NPT, _zero_row, 0)

    def scale(b, j):
        for g in range(B // L):
            nrm = norm_v[j, pl.ds(g * L, L)]
            for e in range(L):
                sp = _splat_lane(nrm, e)
                r = g * L + e
                rows[b][r, pl.ds(0, L)] = rows[b][r, pl.ds(0, L)] * sp
                rows[b][r, pl.ds(L, L)] = rows[b][r, pl.ds(L, L)] * sp

    def run_pipe(src_of, acc, j0, jend):
        def gather(b, j):
            pltpu.async_copy(src_of.at[src_v.at[j]], rows[b], gsem[b])

        def scatter(b, j):
            pltpu.async_copy(rows[b], acc.at[dst_v.at[j]], ssem[b], add=True)

        def wait_gather(b, j):
            pltpu.make_async_copy(src_of.at[src_v.at[j]], rows[b],
                                  gsem[b]).wait()

        def wait_scatter(b, j):
            pltpu.make_async_copy(rows[b], acc.at[dst_v.at[j]],
                                  ssem[b]).wait()

        for b in range(NBUF):
            gather(b, j0 + b)

        def quad(jq, carry):
            j = j0 + NBUF * jq
            for b in range(NBUF):
                jj = j + b
                wait_gather(b, jj)
                scale(b, jj)
                scatter(b, jj)
            for b in range(NBUF):
                jn = jnp.minimum(j + NBUF + b, jend - 1)
                wait_scatter(b, j + b)
                gather(b, jn)
            return carry

        lax.fori_loop(0, (jend - j0) // NBUF, quad, 0)
        for b in range(NBUF):
            wait_gather(b, jend - 1)

    def one_hop(cur, acc, kidx):
        pltpu.sync_copy(zero_v, acc.at[pl.ds(sid * NPT, NPT)])
        plsc.subcore_barrier()
        run_pipe(cur, acc, 0, NCHUNK)
        plsc.subcore_barrier()
        pltpu.sync_copy(acc.at[pl.ds(sid * NPT, NPT)],
                        out.at[kidx, pl.ds(sid * NPT, NPT),
                               pl.ds(cid * CH, CH)])

    def two_hops(i, carry):
        one_hop(bufA, bufB, 2 * i)
        one_hop(bufB, bufA, 2 * i + 1)
        return carry

    lax.fori_loop(0, K // 2, two_hops, 0)


def _prop(hcol, srcs, dsts, norms):
    mesh = plsc.VectorSubcoreMesh(core_axis_name="c", subcore_axis_name="s",
                                  num_cores=NC, num_subcores=NS)
    return pl.kernel(
        _prop_body,
        out_type=jax.ShapeDtypeStruct((K, N_PAD, C), jnp.float32),
        mesh=mesh,
        compiler_params=pltpu.CompilerParams(use_tc_tiling_on_sc=False),
        scratch_types=[
            pltpu.VMEM((NCHUNK, B), jnp.int32),
            pltpu.VMEM((NCHUNK, B), jnp.int32),
            pltpu.VMEM((NCHUNK, B), jnp.float32),
            pltpu.VMEM((NPT, CH), jnp.float32),
            pltpu.VMEM_SHARED((N_PAD, CH), jnp.float32),
            pltpu.VMEM_SHARED((N_PAD, CH), jnp.float32),
        ] + [pltpu.VMEM((B, CH), jnp.float32)] * NBUF
          + [pltpu.SemaphoreType.DMA] * (2 * NBUF),
    )(hcol, srcs, dsts, norms)


ROWS_BLK = 1280


def _final_body(hc_ref, p_ref, w_ref, bp_ref, out_ref):
    w = w_ref[...][0]
    wlo = w[:CH]
    whi = w[CH:]
    b = bp_ref[0, 0]
    h_lo = hc_ref[0]
    h_hi = hc_ref[1]
    s0 = jax.nn.sigmoid(
        jnp.sum(h_lo * wlo[None, :], axis=1)
        + jnp.sum(h_hi * whi[None, :], axis=1) + b)
    acc = jnp.concatenate([s0[:, None] * h_lo, s0[:, None] * h_hi], axis=1)
    p = p_ref[...]
    for k in range(K):
        lk = jnp.sum(p[k] * w[None, :], axis=1) + b
        sk = jax.nn.sigmoid(lk)
        acc = acc + sk[:, None] * p[k]
    out = acc
    m = jnp.max(out, axis=1, keepdims=True)
    ex = jnp.exp(out - m)
    out_ref[...] = out - m - jnp.log(jnp.sum(ex, axis=1, keepdims=True))


def _final(hc, preds, w2d, bp):
    grid = N_PAD // ROWS_BLK
    return pl.pallas_call(
        _final_body,
        grid=(grid,),
        in_specs=[
            pl.BlockSpec((NC, ROWS_BLK, CH), lambda i: (0, i, 0)),
            pl.BlockSpec((K, ROWS_BLK, C), lambda i: (0, i, 0)),
            pl.BlockSpec((1, C), lambda i: (0, 0)),
            pl.BlockSpec((1, 1), lambda i: (0, 0)),
        ],
        out_specs=pl.BlockSpec((ROWS_BLK, C), lambda i: (i, 0)),
        out_shape=jax.ShapeDtypeStruct((N_PAD, C), jnp.float32),
        compiler_params=pltpu.CompilerParams(
            vmem_limit_bytes=100 * 1024 * 1024),
    )(hc, preds, w2d, bp)


def kernel(x, edge_index, norm, W1, b1, gamma, beta, W2, b2, w_proj, b_proj):
    hcol = _mlp(x, W1, b1, gamma, beta, W2, b2)

    pad = E_PAD - E
    src = jnp.concatenate([edge_index[0], jnp.zeros((pad,), jnp.int32)])
    dst = jnp.concatenate([edge_index[1], jnp.zeros((pad,), jnp.int32)])
    nrm = jnp.concatenate([norm, jnp.zeros((pad,), jnp.float32)])
    srcs = src.reshape(NS, NCHUNK, B)
    dsts = dst.reshape(NS, NCHUNK, B)
    norms = nrm.reshape(NS, NCHUNK, B)

    preds = _prop(hcol, srcs, dsts, norms)

    out = _final(hcol, preds, w_proj.reshape(1, C),
                 jnp.reshape(b_proj, (1, 1)))
    return out[:N]


# speedup vs baseline: 1.6337x; 1.0008x over previous
import functools

import jax
import jax.numpy as jnp
from jax import lax
from jax.experimental import pallas as pl
from jax.experimental.pallas import tpu as pltpu
from jax.experimental.pallas import tpu_sc as plsc

N = 10000
E = 320000
F_IN = 128
HID = 128
C = 64
K = 10

NC = 2
NS = 16
L = 16
CH = C // NC
B = 128
NBUF = 2
NCHUNK = 158
EPT = NCHUNK * B
E_PAD = EPT * NS
N_PAD = 10240
NPT = N_PAD // NS


_SPLAT_DN = lax.GatherDimensionNumbers(
    offset_dims=(), collapsed_slice_dims=(0,), start_index_map=(0,))


def _splat_lane(vec, e):
    idx = jnp.full((L, 1), e, jnp.int32)
    return lax.gather(vec, idx, _SPLAT_DN, (1,),
                      mode=lax.GatherScatterMode.PROMISE_IN_BOUNDS)


def _mlp_body(x_ref, w1_ref, b1_ref, g_ref, be_ref, w2_ref, b2_ref, h_ref):
    h1 = jnp.dot(x_ref[...], w1_ref[...], preferred_element_type=jnp.float32)
    h1 = h1 + b1_ref[...][None, :]
    mu = jnp.mean(h1, axis=0, keepdims=True)
    var = jnp.mean((h1 - mu) ** 2, axis=0, keepdims=True)
    hn = (h1 - mu) * lax.rsqrt(var + 1e-5)
    hn = hn * g_ref[...][None, :] + be_ref[...][None, :]
    hr = jnp.maximum(hn, 0.0)
    h = (jnp.dot(hr, w2_ref[...], preferred_element_type=jnp.float32)
         + b2_ref[...][None, :])
    hp = jnp.concatenate(
        [h, jnp.zeros((N_PAD - N, C), jnp.float32)], axis=0)
    h_ref[...] = jnp.stack([hp[:, :CH], hp[:, CH:]], axis=0)


def _mlp(x, W1, b1, gamma, beta, W2, b2):
    return pl.pallas_call(
        _mlp_body,
        out_shape=jax.ShapeDtypeStruct((NC, N_PAD, CH), jnp.float32),
    )(x, W1, b1, gamma, beta, W2, b2)


def _prop_body(hcol, srcs, dsts, norms, out, src_v, dst_v, norm_v, zero_v,
               bufA, bufB, *rows_and_sems):
    rows = rows_and_sems[:NBUF]
    gsem = rows_and_sems[NBUF:2 * NBUF]
    ssem = rows_and_sems[2 * NBUF:3 * NBUF]
    cid = lax.axis_index("c")
    sid = lax.axis_index("s")

    pltpu.sync_copy(srcs.at[sid], src_v)
    pltpu.sync_copy(dsts.at[sid], dst_v)
    pltpu.sync_copy(norms.at[sid], norm_v)

    pltpu.sync_copy(hcol.at[cid, pl.ds(sid * NPT, NPT)],
                    bufA.at[pl.ds(sid * NPT, NPT)])

    zvec = jnp.zeros((L,), jnp.float32)

    def _zero_row(r, _):
        zero_v[r, pl.ds(0, L)] = zvec
        zero_v[r, pl.ds(L, L)] = zvec
        return 0

    lax.fori_loop(0, NPT, _zero_row, 0)

    def scale(b, j):
        for g in range(B // L):
            nrm = norm_v[j, pl.ds(g * L, L)]
            for e in range(L):
                sp = _splat_lane(nrm, e)
                r = g * L + e
                rows[b][r, pl.ds(0, L)] = rows[b][r, pl.ds(0, L)] * sp
                rows[b][r, pl.ds(L, L)] = rows[b][r, pl.ds(L, L)] * sp

    def run_pipe(src_of, acc, j0, jend):
        def gather(b, j):
            pltpu.async_copy(src_of.at[src_v.at[j]], rows[b], gsem[b])

        def scatter(b, j):
            pltpu.async_copy(rows[b], acc.at[dst_v.at[j]], ssem[b], add=True)

        def wait_gather(b, j):
            pltpu.make_async_copy(src_of.at[src_v.at[j]], rows[b],
                                  gsem[b]).wait()

        def wait_scatter(b, j):
            pltpu.make_async_copy(rows[b], acc.at[dst_v.at[j]],
                                  ssem[b]).wait()

        for b in range(NBUF):
            gather(b, j0 + b)

        def quad(jq, carry):
            j = j0 + NBUF * jq
            for b in range(NBUF):
                jj = j + b
                wait_gather(b, jj)
                scale(b, jj)
                scatter(b, jj)
            for b in range(NBUF):
                jn = jnp.minimum(j + NBUF + b, jend - 1)
                wait_scatter(b, j + b)
                gather(b, jn)
            return carry

        lax.fori_loop(0, (jend - j0) // NBUF, quad, 0)
        for b in range(NBUF):
            wait_gather(b, jend - 1)

    def one_hop(cur, acc, kidx):
        pltpu.sync_copy(zero_v, acc.at[pl.ds(sid * NPT, NPT)])
        plsc.subcore_barrier()
        run_pipe(cur, acc, 0, NCHUNK)
        plsc.subcore_barrier()
        pltpu.sync_copy(acc.at[pl.ds(sid * NPT, NPT)],
                        out.at[kidx, pl.ds(sid * NPT, NPT),
                               pl.ds(cid * CH, CH)])

    def two_hops(i, carry):
        one_hop(bufA, bufB, 2 * i)
        one_hop(bufB, bufA, 2 * i + 1)
        return carry

    lax.fori_loop(0, K // 2, two_hops, 0)


def _prop(hcol, srcs, dsts, norms):
    mesh = plsc.VectorSubcoreMesh(core_axis_name="c", subcore_axis_name="s",
                                  num_cores=NC, num_subcores=NS)
    return pl.kernel(
        _prop_body,
        out_type=jax.ShapeDtypeStruct((K, N_PAD, C), jnp.float32),
        mesh=mesh,
        compiler_params=pltpu.CompilerParams(use_tc_tiling_on_sc=False),
        scratch_types=[
            pltpu.VMEM((NCHUNK, B), jnp.int32),
            pltpu.VMEM((NCHUNK, B), jnp.int32),
            pltpu.VMEM((NCHUNK, B), jnp.float32),
            pltpu.VMEM((NPT, CH), jnp.float32),
            pltpu.VMEM_SHARED((N_PAD, CH), jnp.float32),
            pltpu.VMEM_SHARED((N_PAD, CH), jnp.float32),
        ] + [pltpu.VMEM((B, CH), jnp.float32)] * NBUF
          + [pltpu.SemaphoreType.DMA] * (2 * NBUF),
    )(hcol, srcs, dsts, norms)


ROWS_BLK = 2560


def _final_body(hc_ref, p_ref, w_ref, bp_ref, out_ref):
    w = w_ref[...][0]
    wlo = w[:CH]
    whi = w[CH:]
    b = bp_ref[0, 0]
    h_lo = hc_ref[0]
    h_hi = hc_ref[1]
    s0 = jax.nn.sigmoid(
        jnp.sum(h_lo * wlo[None, :], axis=1)
        + jnp.sum(h_hi * whi[None, :], axis=1) + b)
    acc = jnp.concatenate([s0[:, None] * h_lo, s0[:, None] * h_hi], axis=1)
    p = p_ref[...]
    for k in range(K):
        lk = jnp.sum(p[k] * w[None, :], axis=1) + b
        sk = jax.nn.sigmoid(lk)
        acc = acc + sk[:, None] * p[k]
    out = acc
    m = jnp.max(out, axis=1, keepdims=True)
    ex = jnp.exp(out - m)
    out_ref[...] = out - m - jnp.log(jnp.sum(ex, axis=1, keepdims=True))


def _final(hc, preds, w2d, bp):
    grid = N_PAD // ROWS_BLK
    return pl.pallas_call(
        _final_body,
        grid=(grid,),
        in_specs=[
            pl.BlockSpec((NC, ROWS_BLK, CH), lambda i: (0, i, 0)),
            pl.BlockSpec((K, ROWS_BLK, C), lambda i: (0, i, 0)),
            pl.BlockSpec((1, C), lambda i: (0, 0)),
            pl.BlockSpec((1, 1), lambda i: (0, 0)),
        ],
        out_specs=pl.BlockSpec((ROWS_BLK, C), lambda i: (i, 0)),
        out_shape=jax.ShapeDtypeStruct((N_PAD, C), jnp.float32),
        compiler_params=pltpu.CompilerParams(
            vmem_limit_bytes=100 * 1024 * 1024),
    )(hc, preds, w2d, bp)


def kernel(x, edge_index, norm, W1, b1, gamma, beta, W2, b2, w_proj, b_proj):
    hcol = _mlp(x, W1, b1, gamma, beta, W2, b2)

    pad = E_PAD - E
    src = jnp.concatenate([edge_index[0], jnp.zeros((pad,), jnp.int32)])
    dst = jnp.concatenate([edge_index[1], jnp.zeros((pad,), jnp.int32)])
    nrm = jnp.concatenate([norm, jnp.zeros((pad,), jnp.float32)])
    srcs = src.reshape(NS, NCHUNK, B)
    dsts = dst.reshape(NS, NCHUNK, B)
    norms = nrm.reshape(NS, NCHUNK, B)

    preds = _prop(hcol, srcs, dsts, norms)

    out = _final(hcol, preds, w_proj.reshape(1, C),
                 jnp.reshape(b_proj, (1, 1)))
    return out[:N]


# speedup vs baseline: 1.6916x; 1.0354x over previous
import functools

import jax
import jax.numpy as jnp
from jax import lax
from jax.experimental import pallas as pl
from jax.experimental.pallas import tpu as pltpu
from jax.experimental.pallas import tpu_sc as plsc

N = 10000
E = 320000
F_IN = 128
HID = 128
C = 64
K = 10

NC = 2
NS = 16
L = 16
CH = C // NC
B = 128
NBUF = 2
NCHUNK = 158
EPT = NCHUNK * B
E_PAD = EPT * NS
N_PAD = 10240
NPT = N_PAD // NS


_SPLAT_DN = lax.GatherDimensionNumbers(
    offset_dims=(), collapsed_slice_dims=(0,), start_index_map=(0,))


def _splat_lane(vec, e):
    idx = jnp.full((L, 1), e, jnp.int32)
    return lax.gather(vec, idx, _SPLAT_DN, (1,),
                      mode=lax.GatherScatterMode.PROMISE_IN_BOUNDS)


def _mlp_body(x_ref, w1_ref, b1_ref, g_ref, be_ref, w2_ref, b2_ref, h_ref):
    h1 = jnp.dot(x_ref[...], w1_ref[...], preferred_element_type=jnp.float32)
    h1 = h1 + b1_ref[...][None, :]
    mu = jnp.mean(h1, axis=0, keepdims=True)
    var = jnp.mean((h1 - mu) ** 2, axis=0, keepdims=True)
    hn = (h1 - mu) * lax.rsqrt(var + 1e-5)
    hn = hn * g_ref[...][None, :] + be_ref[...][None, :]
    hr = jnp.maximum(hn, 0.0)
    h = (jnp.dot(hr, w2_ref[...], preferred_element_type=jnp.float32)
         + b2_ref[...][None, :])
    hp = jnp.concatenate(
        [h, jnp.zeros((N_PAD - N, C), jnp.float32)], axis=0)
    h_ref[...] = jnp.stack([hp[:, :CH], hp[:, CH:]], axis=0)


def _mlp(x, W1, b1, gamma, beta, W2, b2):
    return pl.pallas_call(
        _mlp_body,
        out_shape=jax.ShapeDtypeStruct((NC, N_PAD, CH), jnp.float32),
    )(x, W1, b1, gamma, beta, W2, b2)


def _prop_body(hcol, srcs, dsts, norms, out, src_v, dst_v, norm_v, zero_v,
               bufA, bufB, *rows_and_sems):
    rows = rows_and_sems[:NBUF]
    gsem = rows_and_sems[NBUF:2 * NBUF]
    ssem = rows_and_sems[2 * NBUF:3 * NBUF]
    cid = lax.axis_index("c")
    sid = lax.axis_index("s")

    pltpu.sync_copy(srcs.at[sid], src_v)
    pltpu.sync_copy(dsts.at[sid], dst_v)
    pltpu.sync_copy(norms.at[sid], norm_v)

    pltpu.sync_copy(hcol.at[cid, pl.ds(sid * NPT, NPT)],
                    bufA.at[pl.ds(sid * NPT, NPT)])

    zvec = jnp.zeros((L,), jnp.float32)

    def _zero_row(r, _):
        zero_v[r, pl.ds(0, L)] = zvec
        zero_v[r, pl.ds(L, L)] = zvec
        return 0

    lax.fori_loop(0, NPT, _zero_row, 0)

    def scale(b, j):
        for g in range(B // L):
            nrm = norm_v[j, pl.ds(g * L, L)]
            for e in range(L):
                sp = _splat_lane(nrm, e)
                r = g * L + e
                rows[b][r, pl.ds(0, L)] = rows[b][r, pl.ds(0, L)] * sp
                rows[b][r, pl.ds(L, L)] = rows[b][r, pl.ds(L, L)] * sp

    def run_pipe(src_of, acc, j0, jend):
        def gather(b, j):
            pltpu.async_copy(src_of.at[src_v.at[j]], rows[b], gsem[b])

        def scatter(b, j):
            pltpu.async_copy(rows[b], acc.at[dst_v.at[j]], ssem[b], add=True)

        def wait_gather(b, j):
            pltpu.make_async_copy(src_of.at[src_v.at[j]], rows[b],
                                  gsem[b]).wait()

        def wait_scatter(b, j):
            pltpu.make_async_copy(rows[b], acc.at[dst_v.at[j]],
                                  ssem[b]).wait()

        for b in range(NBUF):
            gather(b, j0 + b)

        def quad(jq, carry):
            j = j0 + NBUF * jq
            for b in range(NBUF):
                jj = j + b
                wait_gather(b, jj)
                scale(b, jj)
                scatter(b, jj)
            for b in range(NBUF):
                jn = jnp.minimum(j + NBUF + b, jend - 1)
                wait_scatter(b, j + b)
                gather(b, jn)
            return carry

        lax.fori_loop(0, (jend - j0) // NBUF, quad, 0)
        for b in range(NBUF):
            wait_gather(b, jend - 1)

    def one_hop(cur, acc, kidx):
        pltpu.sync_copy(zero_v, acc.at[pl.ds(sid * NPT, NPT)])
        plsc.subcore_barrier()
        run_pipe(cur, acc, 0, NCHUNK)
        plsc.subcore_barrier()
        pltpu.sync_copy(acc.at[pl.ds(sid * NPT, NPT)],
                        out.at[kidx, pl.ds(sid * NPT, NPT),
                               pl.ds(cid * CH, CH)])

    def two_hops(i, carry):
        one_hop(bufA, bufB, 2 * i)
        one_hop(bufB, bufA, 2 * i + 1)
        return carry

    lax.fori_loop(0, K // 2, two_hops, 0)


def _prop(hcol, srcs, dsts, norms):
    mesh = plsc.VectorSubcoreMesh(core_axis_name="c", subcore_axis_name="s",
                                  num_cores=NC, num_subcores=NS)
    return pl.kernel(
        _prop_body,
        out_type=jax.ShapeDtypeStruct((K, N_PAD, C), jnp.float32),
        mesh=mesh,
        compiler_params=pltpu.CompilerParams(use_tc_tiling_on_sc=False),
        scratch_types=[
            pltpu.VMEM((NCHUNK, B), jnp.int32),
            pltpu.VMEM((NCHUNK, B), jnp.int32),
            pltpu.VMEM((NCHUNK, B), jnp.float32),
            pltpu.VMEM((NPT, CH), jnp.float32),
            pltpu.VMEM_SHARED((N_PAD, CH), jnp.float32),
            pltpu.VMEM_SHARED((N_PAD, CH), jnp.float32),
        ] + [pltpu.VMEM((B, CH), jnp.float32)] * NBUF
          + [pltpu.SemaphoreType.DMA] * (2 * NBUF),
    )(hcol, srcs, dsts, norms)


ROWS_BLK = 2560
R2 = ROWS_BLK // 2


def _final_body(hp_ref, p_ref, w_ref, bp_ref, out_ref):
    w = w_ref[...][0]
    b = bp_ref[0, 0]
    hp = hp_ref[...]

    def score(v):
        return jax.nn.sigmoid(jnp.sum(v * w[None, :], axis=1) + b)

    he = hp[:, :C]
    ho = hp[:, C:]
    acce = score(he)[:, None] * he
    acco = score(ho)[:, None] * ho
    p = p_ref[...]
    for k in range(K):
        pe = p[k][:, :C]
        po = p[k][:, C:]
        acce = acce + score(pe)[:, None] * pe
        acco = acco + score(po)[:, None] * po

    def lsm(v):
        m = jnp.max(v, axis=1, keepdims=True)
        return v - m - jnp.log(jnp.sum(jnp.exp(v - m), axis=1, keepdims=True))

    out_ref[...] = jnp.concatenate([lsm(acce), lsm(acco)], axis=1)


def _final(hpair, preds, w2d, bp):
    grid = N_PAD // ROWS_BLK
    p2 = preds.reshape(K, N_PAD // 2, 2 * C)
    out = pl.pallas_call(
        _final_body,
        grid=(grid,),
        in_specs=[
            pl.BlockSpec((R2, 2 * C), lambda i: (i, 0)),
            pl.BlockSpec((K, R2, 2 * C), lambda i: (0, i, 0)),
            pl.BlockSpec((1, C), lambda i: (0, 0)),
            pl.BlockSpec((1, 1), lambda i: (0, 0)),
        ],
        out_specs=pl.BlockSpec((R2, 2 * C), lambda i: (i, 0)),
        out_shape=jax.ShapeDtypeStruct((N_PAD // 2, 2 * C), jnp.float32),
        compiler_params=pltpu.CompilerParams(
            vmem_limit_bytes=100 * 1024 * 1024),
    )(hpair, p2, w2d, bp)
    return out.reshape(N_PAD, C)


def kernel(x, edge_index, norm, W1, b1, gamma, beta, W2, b2, w_proj, b_proj):
    hcol = _mlp(x, W1, b1, gamma, beta, W2, b2)

    pad = E_PAD - E
    src = jnp.concatenate([edge_index[0], jnp.zeros((pad,), jnp.int32)])
    dst = jnp.concatenate([edge_index[1], jnp.zeros((pad,), jnp.int32)])
    nrm = jnp.concatenate([norm, jnp.zeros((pad,), jnp.float32)])
    srcs = src.reshape(NS, NCHUNK, B)
    dsts = dst.reshape(NS, NCHUNK, B)
    norms = nrm.reshape(NS, NCHUNK, B)

    preds = _prop(hcol, srcs, dsts, norms)

    hpair = jnp.concatenate([hcol[0], hcol[1]],
                            axis=1).reshape(N_PAD // 2, 2 * C)
    out = _final(hpair, preds, w_proj.reshape(1, C),
                 jnp.reshape(b_proj, (1, 1)))
    return out[:N]


# speedup vs baseline: 1.7368x; 1.0267x over previous
import functools

import jax
import jax.numpy as jnp
from jax import lax
from jax.experimental import pallas as pl
from jax.experimental.pallas import tpu as pltpu
from jax.experimental.pallas import tpu_sc as plsc

N = 10000
E = 320000
F_IN = 128
HID = 128
C = 64
K = 10

NC = 2
NS = 16
L = 16
CH = C // NC
B = 128
NBUF = 2
NCHUNK = 158
EPT = NCHUNK * B
E_PAD = EPT * NS
N_PAD = 10240
NPT = N_PAD // NS


_SPLAT_DN = lax.GatherDimensionNumbers(
    offset_dims=(), collapsed_slice_dims=(0,), start_index_map=(0,))


def _splat_lane(vec, e):
    idx = jnp.full((L, 1), e, jnp.int32)
    return lax.gather(vec, idx, _SPLAT_DN, (1,),
                      mode=lax.GatherScatterMode.PROMISE_IN_BOUNDS)


def _mlp_body(x_ref, w1_ref, b1_ref, g_ref, be_ref, w2_ref, b2_ref, h_ref):
    h1 = jnp.dot(x_ref[...], w1_ref[...], preferred_element_type=jnp.float32)
    h1 = h1 + b1_ref[...][None, :]
    mu = jnp.mean(h1, axis=0, keepdims=True)
    var = jnp.mean((h1 - mu) ** 2, axis=0, keepdims=True)
    hn = (h1 - mu) * lax.rsqrt(var + 1e-5)
    hn = hn * g_ref[...][None, :] + be_ref[...][None, :]
    hr = jnp.maximum(hn, 0.0)
    h = (jnp.dot(hr, w2_ref[...], preferred_element_type=jnp.float32)
         + b2_ref[...][None, :])
    hp = jnp.concatenate(
        [h, jnp.zeros((N_PAD - N, C), jnp.float32)], axis=0)
    h_ref[...] = jnp.stack([hp[:, :CH], hp[:, CH:]], axis=0)


def _mlp(x, W1, b1, gamma, beta, W2, b2):
    return pl.pallas_call(
        _mlp_body,
        out_shape=jax.ShapeDtypeStruct((NC, N_PAD, CH), jnp.float32),
    )(x, W1, b1, gamma, beta, W2, b2)


def _prop_body(hcol, srcs, dsts, norms, out, src_v, dst_v, norm_v, zero_v,
               bufA, bufB, osemA, osemB, *rows_and_sems):
    rows = rows_and_sems[:NBUF]
    gsem = rows_and_sems[NBUF:2 * NBUF]
    ssem = rows_and_sems[2 * NBUF:3 * NBUF]
    cid = lax.axis_index("c")
    sid = lax.axis_index("s")

    pltpu.sync_copy(srcs.at[sid], src_v)
    pltpu.sync_copy(dsts.at[sid], dst_v)
    pltpu.sync_copy(norms.at[sid], norm_v)

    pltpu.sync_copy(hcol.at[cid, pl.ds(sid * NPT, NPT)],
                    bufA.at[pl.ds(sid * NPT, NPT)])

    zvec = jnp.zeros((L,), jnp.float32)

    def _zero_row(r, _):
        zero_v[r, pl.ds(0, L)] = zvec
        zero_v[r, pl.ds(L, L)] = zvec
        return 0

    lax.fori_loop(0, NPT, _zero_row, 0)

    def scale(b, j):
        for g in range(B // L):
            nrm = norm_v[j, pl.ds(g * L, L)]
            for e in range(L):
                sp = _splat_lane(nrm, e)
                r = g * L + e
                rows[b][r, pl.ds(0, L)] = rows[b][r, pl.ds(0, L)] * sp
                rows[b][r, pl.ds(L, L)] = rows[b][r, pl.ds(L, L)] * sp

    def run_pipe(src_of, acc, j0, jend):
        def gather(b, j):
            pltpu.async_copy(src_of.at[src_v.at[j]], rows[b], gsem[b])

        def scatter(b, j):
            pltpu.async_copy(rows[b], acc.at[dst_v.at[j]], ssem[b], add=True)

        def wait_gather(b, j):
            pltpu.make_async_copy(src_of.at[src_v.at[j]], rows[b],
                                  gsem[b]).wait()

        def wait_scatter(b, j):
            pltpu.make_async_copy(rows[b], acc.at[dst_v.at[j]],
                                  ssem[b]).wait()

        for b in range(NBUF):
            gather(b, j0 + b)

        def quad(jq, carry):
            j = j0 + NBUF * jq
            for b in range(NBUF):
                jj = j + b
                wait_gather(b, jj)
                scale(b, jj)
                scatter(b, jj)
            for b in range(NBUF):
                jn = jnp.minimum(j + NBUF + b, jend - 1)
                wait_scatter(b, j + b)
                gather(b, jn)
            return carry

        lax.fori_loop(0, (jend - j0) // NBUF, quad, 0)
        for b in range(NBUF):
            wait_gather(b, jend - 1)

    def out_slice(kidx):
        return out.at[kidx, pl.ds(sid * NPT, NPT), pl.ds(cid * CH, CH)]

    def one_hop(cur, acc, kidx, osem, wait_prev):
        @pl.when(wait_prev)
        def _():
            pltpu.make_async_copy(acc.at[pl.ds(sid * NPT, NPT)],
                                  out_slice(kidx), osem).wait()

        pltpu.sync_copy(zero_v, acc.at[pl.ds(sid * NPT, NPT)])
        plsc.subcore_barrier()
        run_pipe(cur, acc, 0, NCHUNK)
        plsc.subcore_barrier()
        pltpu.async_copy(acc.at[pl.ds(sid * NPT, NPT)], out_slice(kidx),
                         osem)

    def two_hops(i, carry):
        wp = i > 0
        one_hop(bufA, bufB, 2 * i, osemB, wp)
        one_hop(bufB, bufA, 2 * i + 1, osemA, wp)
        return carry

    lax.fori_loop(0, K // 2, two_hops, 0)
    pltpu.make_async_copy(bufB.at[pl.ds(sid * NPT, NPT)],
                          out_slice(K - 2), osemB).wait()
    pltpu.make_async_copy(bufA.at[pl.ds(sid * NPT, NPT)],
                          out_slice(K - 1), osemA).wait()


def _prop(hcol, srcs, dsts, norms):
    mesh = plsc.VectorSubcoreMesh(core_axis_name="c", subcore_axis_name="s",
                                  num_cores=NC, num_subcores=NS)
    return pl.kernel(
        _prop_body,
        out_type=jax.ShapeDtypeStruct((K, N_PAD, C), jnp.float32),
        mesh=mesh,
        compiler_params=pltpu.CompilerParams(use_tc_tiling_on_sc=False),
        scratch_types=[
            pltpu.VMEM((NCHUNK, B), jnp.int32),
            pltpu.VMEM((NCHUNK, B), jnp.int32),
            pltpu.VMEM((NCHUNK, B), jnp.float32),
            pltpu.VMEM((NPT, CH), jnp.float32),
            pltpu.VMEM_SHARED((N_PAD, CH), jnp.float32),
            pltpu.VMEM_SHARED((N_PAD, CH), jnp.float32),
            pltpu.SemaphoreType.DMA,
            pltpu.SemaphoreType.DMA,
        ] + [pltpu.VMEM((B, CH), jnp.float32)] * NBUF
          + [pltpu.SemaphoreType.DMA] * (2 * NBUF),
    )(hcol, srcs, dsts, norms)


ROWS_BLK = 2560
R2 = ROWS_BLK // 2


def _final_body(hp_ref, p_ref, w_ref, bp_ref, out_ref):
    w = w_ref[...][0]
    b = bp_ref[0, 0]
    hp = hp_ref[...]

    def score(v):
        return jax.nn.sigmoid(jnp.sum(v * w[None, :], axis=1) + b)

    he = hp[:, :C]
    ho = hp[:, C:]
    acce = score(he)[:, None] * he
    acco = score(ho)[:, None] * ho
    p = p_ref[...]
    for k in range(K):
        pe = p[k][:, :C]
        po = p[k][:, C:]
        acce = acce + score(pe)[:, None] * pe
        acco = acco + score(po)[:, None] * po

    def lsm(v):
        m = jnp.max(v, axis=1, keepdims=True)
        return v - m - jnp.log(jnp.sum(jnp.exp(v - m), axis=1, keepdims=True))

    out_ref[...] = jnp.concatenate([lsm(acce), lsm(acco)], axis=1)


def _final(hpair, preds, w2d, bp):
    grid = N_PAD // ROWS_BLK
    p2 = preds.reshape(K, N_PAD // 2, 2 * C)
    out = pl.pallas_call(
        _final_body,
        grid=(grid,),
        in_specs=[
            pl.BlockSpec((R2, 2 * C), lambda i: (i, 0)),
            pl.BlockSpec((K, R2, 2 * C), lambda i: (0, i, 0)),
            pl.BlockSpec((1, C), lambda i: (0, 0)),
            pl.BlockSpec((1, 1), lambda i: (0, 0)),
        ],
        out_specs=pl.BlockSpec((R2, 2 * C), lambda i: (i, 0)),
        out_shape=jax.ShapeDtypeStruct((N_PAD // 2, 2 * C), jnp.float32),
        compiler_params=pltpu.CompilerParams(
            vmem_limit_bytes=100 * 1024 * 1024),
    )(hpair, p2, w2d, bp)
    return out.reshape(N_PAD, C)


def kernel(x, edge_index, norm, W1, b1, gamma, beta, W2, b2, w_proj, b_proj):
    hcol = _mlp(x, W1, b1, gamma, beta, W2, b2)

    pad = E_PAD - E
    src = jnp.concatenate([edge_index[0], jnp.zeros((pad,), jnp.int32)])
    dst = jnp.concatenate([edge_index[1], jnp.zeros((pad,), jnp.int32)])
    nrm = jnp.concatenate([norm, jnp.zeros((pad,), jnp.float32)])
    srcs = src.reshape(NS, NCHUNK, B)
    dsts = dst.reshape(NS, NCHUNK, B)
    norms = nrm.reshape(NS, NCHUNK, B)

    preds = _prop(hcol, srcs, dsts, norms)

    hpair = jnp.concatenate([hcol[0], hcol[1]],
                            axis=1).reshape(N_PAD // 2, 2 * C)
    out = _final(hpair, preds, w_proj.reshape(1, C),
                 jnp.reshape(b_proj, (1, 1)))
    return out[:N]
